# baseline (speedup 1.0000x reference)
import functools
import math

import jax
import jax.numpy as jnp
from jax import lax
from jax.experimental import pallas as pl
from jax.experimental.pallas import tpu as pltpu
from jax.experimental.pallas import tpu_sc as plsc

EMB = 64
SEQ = 50
NTOK = 16384
VOCAB = 1000000
SCALE = math.sqrt(EMB)
LANES = 16
RND = 10
RS = SEQ // RND


def _emb_lookup(tok_lin, tab_lin, *, nc, ns):
    bsl = NTOK // ns
    epc = EMB // nc
    seg = (VOCAB // ns) & ~63
    tail = VOCAB - ns * seg
    nch = 32
    ch = seg // nch
    mesh = plsc.VectorSubcoreMesh(core_axis_name="c", subcore_axis_name="s")

    @functools.partial(
        pl.kernel,
        out_type=jax.ShapeDtypeStruct((SEQ * EMB * NTOK,), jnp.float32),
        mesh=mesh,
        scratch_types=[
            [pltpu.VMEM((bsl,), jnp.int32) for _ in range(SEQ)],
            [pltpu.VMEM((bsl,), jnp.float32) for _ in range(RS)],
            [pltpu.VMEM((bsl,), jnp.float32) for _ in range(RS)],
            pltpu.VMEM((ch,), jnp.float32),
            pltpu.VMEM((ch,), jnp.float32),
            pltpu.VMEM_SHARED((VOCAB,), jnp.float32),
            pltpu.SemaphoreType.DMA,
            pltpu.SemaphoreType.DMA,
            pltpu.SemaphoreType.DMA,
            pltpu.SemaphoreType.DMA,
            pltpu.SemaphoreType.DMA,
            pltpu.SemaphoreType.DMA,
            pltpu.SemaphoreType.DMA,
            pltpu.SemaphoreType.DMA,
        ],
    )
    def run(tok_hbm, tab_hbm, out_hbm, idx_v, va, vb, cb0, cb1, row_sh,
            sem_ga, sem_gb, sem_wa, sem_wb, sem_ci0, sem_ci1, sem_co0,
            sem_co1):
        cid = lax.axis_index("c")
        sid = lax.axis_index("s")
        b0 = sid * bsl
        e_base = cid * epc
        vals = (va, vb)
        gsems = (sem_ga, sem_gb)
        wsems = (sem_wa, sem_wb)
        cbs = (cb0, cb1)
        cis = (sem_ci0, sem_ci1)
        cos = (sem_co0, sem_co1)

        def stage_tail(e_row):
            @pl.when(sid == 0)
            def _():
                pltpu.sync_copy(
                    tab_hbm.at[pl.ds(e_row * VOCAB + ns * seg, tail)],
                    cb0.at[pl.ds(0, tail)])
                pltpu.sync_copy(cb0.at[pl.ds(0, tail)],
                                row_sh.at[pl.ds(ns * seg, tail)])

        def stage_row(e_row):
            base = e_row * VOCAB + sid * seg
            hin = [None] * nch
            hout = [None] * nch
            hin[0] = pltpu.async_copy(tab_hbm.at[pl.ds(base, ch)], cb0, cis[0])
            hin[1] = pltpu.async_copy(tab_hbm.at[pl.ds(base + ch, ch)], cb1,
                                      cis[1])
            for k in range(nch):
                b = k & 1
                hin[k].wait()
                hout[k] = pltpu.async_copy(
                    cbs[b], row_sh.at[pl.ds(sid * seg + k * ch, ch)], cos[b])
                if k + 2 < nch:
                    hout[k].wait()
                    hin[k + 2] = pltpu.async_copy(
                        tab_hbm.at[pl.ds(base + (k + 2) * ch, ch)],
                        cbs[b], cis[b])
            hout[nch - 2].wait()
            hout[nch - 1].wait()
            stage_tail(e_row)

        def issue_gathers(r, b):
            hs = []
            for j in range(RS):
                s = r * RS + j
                hs.append(pltpu.async_copy(
                    row_sh.at[idx_v[s]], vals[b][j], gsems[b]))
            return hs

        def drain_writes(b):
            for j in range(RS):
                pltpu.make_async_copy(
                    tab_hbm.at[pl.ds(0, bsl)], vals[b][j], wsems[b]).wait()

        def scale_buf(b):
            def body(k, carry):
                for j in range(RS):
                    vbuf = vals[b][j]
                    for u in range(4):
                        sl = pl.ds((k * 4 + u) * LANES, LANES)
                        vbuf[sl] = vbuf[sl] * SCALE
                return carry
            lax.fori_loop(0, bsl // (4 * LANES), body, 0)

        def issue_writes(r, b, e):
            for j in range(RS):
                s = r * RS + j
                pltpu.async_copy(
                    vals[b][j],
                    out_hbm.at[pl.ds((s * EMB + e) * NTOK + b0, bsl)],
                    wsems[b])

        for s in range(SEQ):
            pltpu.async_copy(tok_hbm.at[pl.ds(s * NTOK + b0, bsl)],
                             idx_v[s], sem_wa)
        for k in range(nch):
            pltpu.sync_copy(
                tab_hbm.at[pl.ds(e_base * VOCAB + sid * seg + k * ch, ch)],
                cb0)
            pltpu.sync_copy(cb0, row_sh.at[pl.ds(sid * seg + k * ch, ch)])
        stage_tail(e_base)
        for s in range(SEQ):
            pltpu.make_async_copy(tok_hbm.at[pl.ds(s * NTOK + b0, bsl)],
                                  idx_v[s], sem_wa).wait()
        plsc.subcore_barrier()

        def e_step(i, carry):
            e = e_base + i

            @pl.when(i > 0)
            def _():
                drain_writes(0)
            pend = [None, None]
            pend[0] = issue_gathers(0, 0)
            for r in range(RND):
                b = r & 1
                nb = 1 - b
                for h in pend[b]:
                    h.wait()
                scale_buf(b)
                if r + 1 < RND:
                    if r == 0:
                        @pl.when(i > 0)
                        def _():
                            drain_writes(1)
                    else:
                        drain_writes(nb)
                    pend[nb] = issue_gathers(r + 1, nb)
                issue_writes(r, b, e)
            plsc.subcore_barrier()
            stage_row(e_base + jnp.minimum(i + 1, epc - 1))
            plsc.subcore_barrier()
            return carry

        lax.fori_loop(0, epc, e_step, 0)
        drain_writes(0)
        drain_writes(1)

    return run(tok_lin, tab_lin)


def kernel(tokens, table):
    info = plsc.get_sparse_core_info()
    tok_lin = tokens.astype(jnp.int32).T.reshape(SEQ * NTOK)
    tab_lin = table.T.reshape(EMB * VOCAB)
    out = _emb_lookup(tok_lin, tab_lin, nc=info.num_cores, ns=info.num_subcores)
    return out.reshape(SEQ, EMB, NTOK).transpose(2, 0, 1)


# speedup vs baseline: 4.7785x; 4.7785x over previous
import functools
import math

import jax
import jax.numpy as jnp
from jax import lax
from jax.experimental import pallas as pl
from jax.experimental.pallas import tpu as pltpu
from jax.experimental.pallas import tpu_sc as plsc

EMB = 64
SEQ = 50
NTOK = 16384
VOCAB = 1000000
SCALE = math.sqrt(EMB)
LANES = 16
C = 2 * SEQ


def _emb_lookup(tok3, table, *, nc, ns):
    nw = nc * ns
    chunks = tok3.shape[1]
    rows_per_w = chunks * C
    b_per_w = rows_per_w // SEQ
    mesh = plsc.VectorSubcoreMesh(core_axis_name="c", subcore_axis_name="s")

    @functools.partial(
        pl.kernel,
        out_type=jax.ShapeDtypeStruct((NTOK, SEQ, EMB), jnp.float32),
        mesh=mesh,
        scratch_types=[
            pltpu.VMEM((chunks, C), jnp.int32),
            pltpu.VMEM((C, EMB), jnp.float32),
            pltpu.VMEM((C, EMB), jnp.float32),
            pltpu.VMEM((C, EMB), jnp.float32),
            pltpu.VMEM((C, EMB), jnp.float32),
            pltpu.SemaphoreType.DMA,
            pltpu.SemaphoreType.DMA,
            pltpu.SemaphoreType.DMA,
        ],
        compiler_params=pltpu.CompilerParams(use_tc_tiling_on_sc=False),
    )
    def run(tok_hbm, tab_hbm, out_hbm, idx_v, buf0, buf1, wb0, wb1,
            sem0, sem1, sem_w):
        wid = lax.axis_index("s") * nc + lax.axis_index("c")
        base_b = wid * b_per_w
        bufs = (buf0, buf1)
        wbufs = (wb0, wb1)
        sems = (sem0, sem1)

        pltpu.sync_copy(tok_hbm.at[wid], idx_v)

        pltpu.async_copy(tab_hbm.at[idx_v.at[0]], buf0, sem0)
        pltpu.async_copy(tab_hbm.at[idx_v.at[1]], buf1, sem1)

        def scale_rows(src_buf, dst_buf):
            def row(r, carry):
                for k in range(EMB // LANES):
                    sl = pl.ds(k * LANES, LANES)
                    dst_buf[r, sl] = src_buf[r, sl] * SCALE
                return carry
            lax.fori_loop(0, C, row, 0)

        def do_chunk(g, b, *, start_next, first):
            buf = bufs[b]
            wbuf = wbufs[b]
            pltpu.make_async_copy(
                tab_hbm.at[pl.ds(0, C)], buf, sems[b]).wait()
            if not first:
                for j in range(2):
                    pltpu.make_async_copy(
                        tab_hbm.at[pl.ds(0, SEQ)],
                        wbuf.at[pl.ds(j * SEQ, SEQ)], sem_w).wait()
            scale_rows(buf, wbuf)
            if start_next:
                pltpu.async_copy(tab_hbm.at[idx_v.at[g + 2]], buf, sems[b])
            for j in range(2):
                pltpu.async_copy(wbuf.at[pl.ds(j * SEQ, SEQ)],
                                 out_hbm.at[base_b + g * 2 + j], sem_w)

        for b in range(2):
            do_chunk(b, b, start_next=True, first=True)

        def step(g2, carry):
            for b in range(2):
                do_chunk(g2 * 2 + b, b, start_next=True, first=False)
            return carry

        lax.fori_loop(1, chunks // 2 - 1, step, 0)
        for b in range(2):
            do_chunk(chunks - 2 + b, b, start_next=False, first=False)
        for b in range(2):
            for j in range(2):
                pltpu.make_async_copy(
                    tab_hbm.at[pl.ds(0, SEQ)],
                    wbufs[b].at[pl.ds(j * SEQ, SEQ)], sem_w).wait()

    return run(tok3, table)


def kernel(tokens, table):
    info = plsc.get_sparse_core_info()
    nw = info.num_cores * info.num_subcores
    chunks = NTOK * SEQ // (nw * C)
    tok3 = tokens.astype(jnp.int32).reshape(nw, chunks, C)
    return _emb_lookup(tok3, table, nc=info.num_cores, ns=info.num_subcores)


# speedup vs baseline: 4.9370x; 1.0332x over previous
import functools
import math

import jax
import jax.numpy as jnp
from jax import lax
from jax.experimental import pallas as pl
from jax.experimental.pallas import tpu as pltpu
from jax.experimental.pallas import tpu_sc as plsc

EMB = 64
SEQ = 50
NTOK = 16384
VOCAB = 1000000
SCALE = math.sqrt(EMB)
LANES = 16
C = 4 * SEQ


def _emb_lookup(tok3, table, *, nc, ns):
    nw = nc * ns
    chunks = tok3.shape[1]
    rows_per_w = chunks * C
    b_per_w = rows_per_w // SEQ
    mesh = plsc.VectorSubcoreMesh(core_axis_name="c", subcore_axis_name="s")

    @functools.partial(
        pl.kernel,
        out_type=jax.ShapeDtypeStruct((NTOK, SEQ, EMB), jnp.float32),
        mesh=mesh,
        scratch_types=[
            pltpu.VMEM((chunks, C), jnp.int32),
            pltpu.VMEM((C, EMB), jnp.float32),
            pltpu.VMEM((C, EMB), jnp.float32),
            pltpu.VMEM((C, EMB), jnp.float32),
            pltpu.VMEM((C, EMB), jnp.float32),
            pltpu.SemaphoreType.DMA,
            pltpu.SemaphoreType.DMA,
            pltpu.SemaphoreType.DMA,
        ],
        compiler_params=pltpu.CompilerParams(use_tc_tiling_on_sc=False),
    )
    def run(tok_hbm, tab_hbm, out_hbm, idx_v, buf0, buf1, wb0, wb1,
            sem0, sem1, sem_w):
        wid = lax.axis_index("s") * nc + lax.axis_index("c")
        base_b = wid * b_per_w
        bufs = (buf0, buf1)
        wbufs = (wb0, wb1)
        sems = (sem0, sem1)

        pltpu.sync_copy(tok_hbm.at[wid], idx_v)

        pltpu.async_copy(tab_hbm.at[idx_v.at[0]], buf0, sem0)
        pltpu.async_copy(tab_hbm.at[idx_v.at[1]], buf1, sem1)

        def scale_rows(src_buf, dst_buf):
            def row(r, carry):
                for k in range(EMB // LANES):
                    sl = pl.ds(k * LANES, LANES)
                    dst_buf[r, sl] = src_buf[r, sl] * SCALE
                return carry
            lax.fori_loop(0, C, row, 0)

        def do_chunk(g, b, *, start_next, first):
            buf = bufs[b]
            wbuf = wbufs[b]
            pltpu.make_async_copy(
                tab_hbm.at[pl.ds(0, C)], buf, sems[b]).wait()
            if not first:
                for j in range(4):
                    pltpu.make_async_copy(
                        tab_hbm.at[pl.ds(0, SEQ)],
                        wbuf.at[pl.ds(j * SEQ, SEQ)], sem_w).wait()
            scale_rows(buf, wbuf)
            if start_next:
                pltpu.async_copy(tab_hbm.at[idx_v.at[g + 2]], buf, sems[b])
            for j in range(4):
                pltpu.async_copy(wbuf.at[pl.ds(j * SEQ, SEQ)],
                                 out_hbm.at[base_b + g * 4 + j], sem_w)

        for b in range(2):
            do_chunk(b, b, start_next=True, first=True)

        def step(g2, carry):
            for b in range(2):
                do_chunk(g2 * 2 + b, b, start_next=True, first=False)
            return carry

        lax.fori_loop(1, chunks // 2 - 1, step, 0)
        for b in range(2):
            do_chunk(chunks - 2 + b, b, start_next=False, first=False)
        for b in range(2):
            for j in range(4):
                pltpu.make_async_copy(
                    tab_hbm.at[pl.ds(0, SEQ)],
                    wbufs[b].at[pl.ds(j * SEQ, SEQ)], sem_w).wait()

    return run(tok3, table)


def kernel(tokens, table):
    info = plsc.get_sparse_core_info()
    nw = info.num_cores * info.num_subcores
    chunks = NTOK * SEQ // (nw * C)
    tok3 = tokens.astype(jnp.int32).reshape(nw, chunks, C)
    return _emb_lookup(tok3, table, nc=info.num_cores, ns=info.num_subcores)


# speedup vs baseline: 4.9652x; 1.0057x over previous
import functools
import math

import jax
import jax.numpy as jnp
from jax import lax
from jax.experimental import pallas as pl
from jax.experimental.pallas import tpu as pltpu
from jax.experimental.pallas import tpu_sc as plsc

EMB = 64
SEQ = 50
NTOK = 16384
VOCAB = 1000000
SCALE = math.sqrt(EMB)
LANES = 16
C = 8 * SEQ


def _emb_lookup(tok3, table, *, nc, ns):
    nw = nc * ns
    chunks = tok3.shape[1]
    rows_per_w = chunks * C
    b_per_w = rows_per_w // SEQ
    mesh = plsc.VectorSubcoreMesh(core_axis_name="c", subcore_axis_name="s")

    @functools.partial(
        pl.kernel,
        out_type=jax.ShapeDtypeStruct((NTOK, SEQ, EMB), jnp.float32),
        mesh=mesh,
        scratch_types=[
            pltpu.VMEM((chunks, C), jnp.int32),
            pltpu.VMEM((C, EMB), jnp.float32),
            pltpu.VMEM((C, EMB), jnp.float32),
            pltpu.VMEM((C, EMB), jnp.float32),
            pltpu.VMEM((C, EMB), jnp.float32),
            pltpu.SemaphoreType.DMA,
            pltpu.SemaphoreType.DMA,
            pltpu.SemaphoreType.DMA,
        ],
        compiler_params=pltpu.CompilerParams(use_tc_tiling_on_sc=False),
    )
    def run(tok_hbm, tab_hbm, out_hbm, idx_v, buf0, buf1, wb0, wb1,
            sem0, sem1, sem_w):
        wid = lax.axis_index("s") * nc + lax.axis_index("c")
        base_b = wid * b_per_w
        bufs = (buf0, buf1)
        wbufs = (wb0, wb1)
        sems = (sem0, sem1)

        pltpu.sync_copy(tok_hbm.at[wid], idx_v)

        pltpu.async_copy(tab_hbm.at[idx_v.at[0]], buf0, sem0)
        pltpu.async_copy(tab_hbm.at[idx_v.at[1]], buf1, sem1)

        def scale_rows(src_buf, dst_buf):
            def row(r, carry):
                for k in range(EMB // LANES):
                    sl = pl.ds(k * LANES, LANES)
                    dst_buf[r, sl] = src_buf[r, sl] * SCALE
                return carry
            lax.fori_loop(0, C, row, 0)

        def do_chunk(g, b, *, start_next, first):
            buf = bufs[b]
            wbuf = wbufs[b]
            pltpu.make_async_copy(
                tab_hbm.at[pl.ds(0, C)], buf, sems[b]).wait()
            if not first:
                for j in range(8):
                    pltpu.make_async_copy(
                        tab_hbm.at[pl.ds(0, SEQ)],
                        wbuf.at[pl.ds(j * SEQ, SEQ)], sem_w).wait()
            scale_rows(buf, wbuf)
            if start_next:
                pltpu.async_copy(tab_hbm.at[idx_v.at[g + 2]], buf, sems[b])
            for j in range(8):
                pltpu.async_copy(wbuf.at[pl.ds(j * SEQ, SEQ)],
                                 out_hbm.at[base_b + g * 8 + j], sem_w)

        for b in range(2):
            do_chunk(b, b, start_next=True, first=True)

        def step(g2, carry):
            for b in range(2):
                do_chunk(g2 * 2 + b, b, start_next=True, first=False)
            return carry

        lax.fori_loop(1, chunks // 2 - 1, step, 0)
        for b in range(2):
            do_chunk(chunks - 2 + b, b, start_next=False, first=False)
        for b in range(2):
            for j in range(8):
                pltpu.make_async_copy(
                    tab_hbm.at[pl.ds(0, SEQ)],
                    wbufs[b].at[pl.ds(j * SEQ, SEQ)], sem_w).wait()

    return run(tok3, table)


def kernel(tokens, table):
    info = plsc.get_sparse_core_info()
    nw = info.num_cores * info.num_subcores
    chunks = NTOK * SEQ // (nw * C)
    tok3 = tokens.astype(jnp.int32).reshape(nw, chunks, C)
    return _emb_lookup(tok3, table, nc=info.num_cores, ns=info.num_subcores)


# speedup vs baseline: 4.9775x; 1.0025x over previous
import functools
import math

import jax
import jax.numpy as jnp
from jax import lax
from jax.experimental import pallas as pl
from jax.experimental.pallas import tpu as pltpu
from jax.experimental.pallas import tpu_sc as plsc

EMB = 64
SEQ = 50
NTOK = 16384
VOCAB = 1000000
SCALE = math.sqrt(EMB)
LANES = 16
C = 8 * SEQ


def _emb_lookup(tok3, table, *, nc, ns):
    nw = nc * ns
    chunks = tok3.shape[1]
    rows_per_w = chunks * C
    b_per_w = rows_per_w // SEQ
    mesh = plsc.VectorSubcoreMesh(core_axis_name="c", subcore_axis_name="s")

    @functools.partial(
        pl.kernel,
        out_type=jax.ShapeDtypeStruct((NTOK, SEQ, EMB), jnp.float32),
        mesh=mesh,
        scratch_types=[
            pltpu.VMEM((chunks, C), jnp.int32),
            pltpu.VMEM((C, EMB), jnp.float32),
            pltpu.VMEM((C, EMB), jnp.float32),
            pltpu.VMEM((C // SEQ, SEQ, EMB), jnp.float32),
            pltpu.VMEM((C // SEQ, SEQ, EMB), jnp.float32),
            pltpu.SemaphoreType.DMA,
            pltpu.SemaphoreType.DMA,
            pltpu.SemaphoreType.DMA,
        ],
        compiler_params=pltpu.CompilerParams(use_tc_tiling_on_sc=False),
    )
    def run(tok_hbm, tab_hbm, out_hbm, idx_v, buf0, buf1, wb0, wb1,
            sem0, sem1, sem_w):
        wid = lax.axis_index("s") * nc + lax.axis_index("c")
        base_b = wid * b_per_w
        bufs = (buf0, buf1)
        wbufs = (wb0, wb1)
        sems = (sem0, sem1)

        pltpu.sync_copy(tok_hbm.at[wid], idx_v)

        pltpu.async_copy(tab_hbm.at[idx_v.at[0]], buf0, sem0)
        pltpu.async_copy(tab_hbm.at[idx_v.at[1]], buf1, sem1)

        def scale_rows(src_buf, dst_buf):
            def row(r, carry):
                for j in range(C // SEQ):
                    for k in range(EMB // LANES):
                        sl = pl.ds(k * LANES, LANES)
                        dst_buf[j, r, sl] = src_buf[j * SEQ + r, sl] * SCALE
                return carry
            lax.fori_loop(0, SEQ, row, 0)

        def do_chunk(g, b, *, start_next, first):
            buf = bufs[b]
            wbuf = wbufs[b]
            pltpu.make_async_copy(
                tab_hbm.at[pl.ds(0, C)], buf, sems[b]).wait()
            if not first:
                pltpu.make_async_copy(
                    out_hbm.at[pl.ds(0, C // SEQ)], wbuf, sem_w).wait()
            scale_rows(buf, wbuf)
            if start_next:
                pltpu.async_copy(tab_hbm.at[idx_v.at[g + 2]], buf, sems[b])
            pltpu.async_copy(
                wbuf, out_hbm.at[pl.ds(base_b + g * (C // SEQ), C // SEQ)],
                sem_w)

        for b in range(2):
            do_chunk(b, b, start_next=True, first=True)

        def step(g2, carry):
            for b in range(2):
                do_chunk(g2 * 2 + b, b, start_next=True, first=False)
            return carry

        lax.fori_loop(1, chunks // 2 - 1, step, 0)
        for b in range(2):
            do_chunk(chunks - 2 + b, b, start_next=False, first=False)
        for b in range(2):
            pltpu.make_async_copy(
                out_hbm.at[pl.ds(0, C // SEQ)], wbufs[b], sem_w).wait()

    return run(tok3, table)


def kernel(tokens, table):
    info = plsc.get_sparse_core_info()
    nw = info.num_cores * info.num_subcores
    chunks = NTOK * SEQ // (nw * C)
    tok3 = tokens.astype(jnp.int32).reshape(nw, chunks, C)
    return _emb_lookup(tok3, table, nc=info.num_cores, ns=info.num_subcores)
